# baseline (speedup 1.0000x reference)
import functools
import jax
import jax.numpy as jnp
from jax import lax
from jax.experimental import pallas as pl
from jax.experimental.pallas import tpu as pltpu
from jax.experimental.pallas import tpu_sc as plsc


def _make_sc_copy(batch, seq, dim, dtype):
    info = plsc.get_sparse_core_info()
    nc, ns = info.num_cores, info.num_subcores
    nw = nc * ns
    rows_w = seq // nw
    ch = min(rows_w, 32)
    nch = rows_w // ch
    mesh = plsc.VectorSubcoreMesh(core_axis_name="c", subcore_axis_name="s")

    @functools.partial(
        pl.kernel,
        out_type=jax.ShapeDtypeStruct((batch, seq, dim), dtype),
        mesh=mesh,
        scratch_types=[
            pltpu.VMEM((ch, dim), dtype),
            pltpu.VMEM((ch, dim), dtype),
            pltpu.SemaphoreType.DMA,
            pltpu.SemaphoreType.DMA,
            pltpu.SemaphoreType.DMA,
            pltpu.SemaphoreType.DMA,
        ],
    )
    def sc_copy(pe_hbm, out_hbm, buf0, buf1, rsem0, rsem1, wsem0, wsem1):
        wid = lax.axis_index("s") * nc + lax.axis_index("c")
        base = wid * rows_w
        bufs = (buf0, buf1)
        rsems = (rsem0, rsem1)
        wsems = (wsem0, wsem1)
        reads = [None] * nch
        writes = [[None] * batch for _ in range(nch)]
        reads[0] = pltpu.async_copy(pe_hbm.at[pl.ds(base, ch)], buf0, rsem0)
        for c in range(nch):
            p = c % 2
            q = (c + 1) % 2
            if c + 1 < nch:
                if c >= 1:
                    for w in writes[c - 1]:
                        w.wait()
                reads[c + 1] = pltpu.async_copy(
                    pe_hbm.at[pl.ds(base + (c + 1) * ch, ch)], bufs[q], rsems[q]
                )
            reads[c].wait()
            for b in range(batch):
                writes[c][b] = pltpu.async_copy(
                    bufs[p], out_hbm.at[b, pl.ds(base + c * ch, ch)], wsems[p]
                )
        for c in (nch - 2, nch - 1):
            if c >= 0:
                for w in writes[c]:
                    w.wait()

    return sc_copy


def kernel(mask, pe):
    batch, seq = mask.shape
    max_len, dim = pe.shape
    return _make_sc_copy(batch, seq, dim, pe.dtype)(pe[:seq])


# speedup vs baseline: 1.5501x; 1.5501x over previous
import jax
import jax.numpy as jnp
from jax.experimental import pallas as pl
from jax.experimental.pallas import tpu as pltpu

_CH = 512


def _body(pe_hbm, out_hbm, buf0, buf1, rs0, rs1, ws0, ws1):
    batch = out_hbm.shape[0]
    seq = pe_hbm.shape[0]
    nch = seq // _CH
    bufs = (buf0, buf1)
    rsems = (rs0, rs1)
    wsems = (ws0, ws1)
    reads = [None] * nch
    writes = [[None] * batch for _ in range(nch)]
    reads[0] = pltpu.make_async_copy(pe_hbm.at[pl.ds(0, _CH)], buf0, rs0)
    reads[0].start()
    for c in range(nch):
        p = c % 2
        q = (c + 1) % 2
        if c + 1 < nch:
            if c >= 1:
                for w in writes[c - 1]:
                    w.wait()
            reads[c + 1] = pltpu.make_async_copy(
                pe_hbm.at[pl.ds((c + 1) * _CH, _CH)], bufs[q], rsems[q]
            )
            reads[c + 1].start()
        reads[c].wait()
        for b in range(batch):
            writes[c][b] = pltpu.make_async_copy(
                bufs[p], out_hbm.at[b, pl.ds(c * _CH, _CH)], wsems[p]
            )
            writes[c][b].start()
    for c in (nch - 2, nch - 1):
        if c >= 0:
            for w in writes[c]:
                w.wait()


def kernel(mask, pe):
    batch, seq = mask.shape
    max_len, dim = pe.shape
    out = pl.pallas_call(
        _body,
        in_specs=[pl.BlockSpec(memory_space=pltpu.HBM)],
        out_specs=pl.BlockSpec(memory_space=pltpu.HBM),
        out_shape=jax.ShapeDtypeStruct((batch, seq, dim), pe.dtype),
        scratch_shapes=[
            pltpu.VMEM((_CH, dim), pe.dtype),
            pltpu.VMEM((_CH, dim), pe.dtype),
            pltpu.SemaphoreType.DMA,
            pltpu.SemaphoreType.DMA,
            pltpu.SemaphoreType.DMA,
            pltpu.SemaphoreType.DMA,
        ],
    )(pe[:seq])
    return out


# speedup vs baseline: 1.7157x; 1.1069x over previous
import jax
import jax.numpy as jnp
from jax.experimental import pallas as pl
from jax.experimental.pallas import tpu as pltpu

_CH = 1024


def _body(pe_hbm, out_hbm, buf0, buf1, rs0, rs1, ws0, ws1):
    batch = out_hbm.shape[0]
    seq = pe_hbm.shape[0]
    nch = seq // _CH
    bufs = (buf0, buf1)
    rsems = (rs0, rs1)
    wsems = (ws0, ws1)
    reads = [None] * nch
    writes = [[None] * batch for _ in range(nch)]
    reads[0] = pltpu.make_async_copy(pe_hbm.at[pl.ds(0, _CH)], buf0, rs0)
    reads[0].start()
    for c in range(nch):
        p = c % 2
        q = (c + 1) % 2
        if c + 1 < nch:
            if c >= 1:
                for w in writes[c - 1]:
                    w.wait()
            reads[c + 1] = pltpu.make_async_copy(
                pe_hbm.at[pl.ds((c + 1) * _CH, _CH)], bufs[q], rsems[q]
            )
            reads[c + 1].start()
        reads[c].wait()
        for b in range(batch):
            writes[c][b] = pltpu.make_async_copy(
                bufs[p], out_hbm.at[b, pl.ds(c * _CH, _CH)], wsems[p]
            )
            writes[c][b].start()
    for c in (nch - 2, nch - 1):
        if c >= 0:
            for w in writes[c]:
                w.wait()


def kernel(mask, pe):
    batch, seq = mask.shape
    max_len, dim = pe.shape
    out = pl.pallas_call(
        _body,
        in_specs=[pl.BlockSpec(memory_space=pltpu.HBM)],
        out_specs=pl.BlockSpec(memory_space=pltpu.HBM),
        out_shape=jax.ShapeDtypeStruct((batch, seq, dim), pe.dtype),
        scratch_shapes=[
            pltpu.VMEM((_CH, dim), pe.dtype),
            pltpu.VMEM((_CH, dim), pe.dtype),
            pltpu.SemaphoreType.DMA,
            pltpu.SemaphoreType.DMA,
            pltpu.SemaphoreType.DMA,
            pltpu.SemaphoreType.DMA,
        ],
    )(pe[:seq])
    return out
